# baseline (speedup 1.0000x reference)
import functools

import jax
import jax.numpy as jnp
from jax import lax
from jax.experimental import pallas as pl
from jax.experimental.pallas import tpu as pltpu
from jax.experimental.pallas import tpu_sc as plsc

N = 16384
DIM = 2048
E = 64
K = 8
L = 16
NW = 32

CH = 2
NCH = N // CH
BA = 1024
BC = 2048
TPW = NCH // NW


def _mm_body(x_ref, w_ref, scores_ref, scores_t_ref):
    logits_t = lax.dot_general(
        w_ref[...], x_ref[...], (((1,), (1,)), ((), ())),
        preferred_element_type=jnp.float32)
    st = jax.nn.sigmoid(logits_t)
    scores_t_ref[...] = st
    scores_ref[...] = st.T


def _sc_route(scores_t_hbm, bias_hbm, tops_t_hbm, idx_t_hbm,
              sc_v, bias_v, tops_v, idx_v):
    c = lax.axis_index("c")
    s = lax.axis_index("s")
    wid = s * 2 + c
    base = wid * TPW
    pltpu.sync_copy(scores_t_hbm.at[:, pl.ds(base, TPW)], sc_v)
    pltpu.sync_copy(bias_hbm, bias_v)

    def half_group(t0):
        keys = [jnp.full((L,), -jnp.inf, jnp.float32) for _ in range(K)]
        idxs = [jnp.zeros((L,), jnp.int32) for _ in range(K)]
        raws = [jnp.zeros((L,), jnp.float32) for _ in range(K)]
        for e in range(E):
            tr = sc_v[e, pl.ds(t0, L)]
            tk = tr + bias_v[e, :]
            ti = jnp.full((L,), e, jnp.int32)
            for j in range(K):
                cs = tk > keys[j]
                keys[j], tk = (jnp.where(cs, tk, keys[j]),
                               jnp.where(cs, keys[j], tk))
                idxs[j], ti = (jnp.where(cs, ti, idxs[j]),
                               jnp.where(cs, idxs[j], ti))
                raws[j], tr = (jnp.where(cs, tr, raws[j]),
                               jnp.where(cs, raws[j], tr))
        denom = raws[0]
        for j in range(1, K):
            denom = denom + raws[j]
        denom = denom + 1e-20
        for j in range(K):
            tops_v[j, pl.ds(t0, L)] = raws[j] / denom
            idx_v[j, pl.ds(t0, L)] = idxs[j]

    def group_body(g, carry):
        half_group(g * (2 * L))
        half_group(g * (2 * L) + L)
        return carry

    lax.fori_loop(0, TPW // (2 * L), group_body, 0)

    pltpu.sync_copy(tops_v, tops_t_hbm.at[:, pl.ds(base, TPW)])
    pltpu.sync_copy(idx_v, idx_t_hbm.at[:, pl.ds(base, TPW)])


def _fin_body(topst_ref, idxt_ref, tops_ref, idx_ref, counts_ref, ent_ref,
              acc_ref):
    i = pl.program_id(0)
    nsteps = pl.num_programs(0)
    tt = topst_ref[...]
    it = idxt_ref[...]
    tops_ref[...] = tt.T
    idx_ref[...] = it.T

    iota64 = lax.broadcasted_iota(jnp.int32, (E, BC), 0)
    part = jnp.zeros((E, BC), jnp.float32)
    for j in range(K):
        part = part + (it[j:j + 1, :] == iota64).astype(jnp.float32)

    ent_part = jnp.sum(tt * jnp.log(tt))

    @pl.when(i == 0)
    def _init():
        acc_ref[...] = part
        ent_ref[...] = jnp.full((1, 1), ent_part, jnp.float32)

    @pl.when(i > 0)
    def _acc():
        acc_ref[...] += part
        ent_ref[...] += ent_part

    @pl.when(i == nsteps - 1)
    def _fin():
        counts_ref[...] = jnp.sum(acc_ref[...], axis=1, keepdims=True)


@jax.jit
def kernel(x, expert_bias, W):
    bias16 = jnp.broadcast_to(expert_bias.reshape(E, 1), (E, L))
    mesh = plsc.VectorSubcoreMesh(core_axis_name="c", subcore_axis_name="s")

    scores_cs, tops_cs, idx_cs, counts_cs, ent_cs = [], [], [], [], []
    for c in range(CH):
        blk0 = c * (NCH // BA)
        scores_c, scores_t_c = pl.pallas_call(
            _mm_body,
            grid=(NCH // BA,),
            in_specs=[
                pl.BlockSpec((BA, DIM), lambda i, b=blk0: (b + i, 0)),
                pl.BlockSpec((E, DIM), lambda i: (0, 0)),
            ],
            out_specs=[
                pl.BlockSpec((BA, E), lambda i: (i, 0)),
                pl.BlockSpec((E, BA), lambda i: (0, i)),
            ],
            out_shape=[
                jax.ShapeDtypeStruct((NCH, E), jnp.float32),
                jax.ShapeDtypeStruct((E, NCH), jnp.float32),
            ],
        )(x, W)
        scores_cs.append(scores_c)

        tops_t_c, idx_t_c = pl.kernel(
            _sc_route,
            mesh=mesh,
            out_type=[
                jax.ShapeDtypeStruct((K, NCH), jnp.float32),
                jax.ShapeDtypeStruct((K, NCH), jnp.int32),
            ],
            scratch_types=[
                pltpu.VMEM((E, TPW), jnp.float32),
                pltpu.VMEM((E, L), jnp.float32),
                pltpu.VMEM((K, TPW), jnp.float32),
                pltpu.VMEM((K, TPW), jnp.int32),
            ],
        )(scores_t_c, bias16)

        tops_c, idx_c, counts_c, ent_c = pl.pallas_call(
            _fin_body,
            grid=(NCH // BC,),
            in_specs=[
                pl.BlockSpec((K, BC), lambda i: (0, i)),
                pl.BlockSpec((K, BC), lambda i: (0, i)),
            ],
            out_specs=[
                pl.BlockSpec((BC, K), lambda i: (i, 0)),
                pl.BlockSpec((BC, K), lambda i: (i, 0)),
                pl.BlockSpec((E, 1), lambda i: (0, 0)),
                pl.BlockSpec((1, 1), lambda i: (0, 0)),
            ],
            out_shape=[
                jax.ShapeDtypeStruct((NCH, K), jnp.float32),
                jax.ShapeDtypeStruct((NCH, K), jnp.int32),
                jax.ShapeDtypeStruct((E, 1), jnp.float32),
                jax.ShapeDtypeStruct((1, 1), jnp.float32),
            ],
            scratch_shapes=[pltpu.VMEM((E, BC), jnp.float32)],
        )(tops_t_c, idx_t_c)
        tops_cs.append(tops_c)
        idx_cs.append(idx_c)
        counts_cs.append(counts_c)
        ent_cs.append(ent_c)

    scores = jnp.concatenate(scores_cs, axis=0)
    tops = jnp.concatenate(tops_cs, axis=0)
    idx = jnp.concatenate(idx_cs, axis=0)
    counts = sum(counts_cs).reshape(E)
    ent = -(sum(ent_cs).reshape(())) * (1.0 / N)
    return (tops, scores, idx, counts, ent)


# speedup vs baseline: 1.8672x; 1.8672x over previous
import functools

import jax
import jax.numpy as jnp
from jax import lax
from jax.experimental import pallas as pl
from jax.experimental.pallas import tpu as pltpu

N = 16384
DIM = 2048
E = 64
K = 8
BN = 2048


def _router_body(x_ref, bias_ref, w_ref, scores_ref, tops_ref, idx_ref,
                 counts_ref, ent_ref):
    i = pl.program_id(0)
    nsteps = pl.num_programs(0)

    logits_t = lax.dot_general(
        w_ref[...], x_ref[...], (((1,), (1,)), ((), ())),
        preferred_element_type=jnp.float32)
    scores_t = jax.nn.sigmoid(logits_t)
    scores_ref[...] = scores_t.T

    biased_t = scores_t + bias_ref[...]
    iota_f = lax.broadcasted_iota(jnp.int32, (E, BN), 0).astype(jnp.float32)

    work = biased_t
    idx_rows = []
    raw_rows = []
    neg_inf = jnp.float32(-jnp.inf)
    for _ in range(K):
        m = jnp.max(work, axis=0, keepdims=True)
        masked_iota = jnp.where(work == m, iota_f, jnp.float32(E))
        idx_f = jnp.min(masked_iota, axis=0, keepdims=True)
        onehot = masked_iota == idx_f
        raw = jnp.sum(jnp.where(onehot, scores_t, 0.0), axis=0, keepdims=True)
        idx_rows.append(idx_f)
        raw_rows.append(raw)
        work = jnp.where(onehot, neg_inf, work)

    sel = (work == neg_inf).astype(jnp.float32)
    counts = jnp.sum(sel, axis=1, keepdims=True)

    idx_t = jnp.concatenate(idx_rows, axis=0)
    raw_t = jnp.concatenate(raw_rows, axis=0)
    denom = jnp.sum(raw_t, axis=0, keepdims=True) + 1e-20
    tops_t = raw_t / denom
    idx_ref[...] = idx_t.T.astype(jnp.int32)
    tops_ref[...] = tops_t.T

    ent_part = jnp.sum(tops_t * jnp.log(tops_t))

    @pl.when(i == 0)
    def _init():
        counts_ref[...] = counts
        ent_ref[...] = jnp.full((1, 1), ent_part, jnp.float32)

    @pl.when(i > 0)
    def _acc():
        counts_ref[...] += counts
        ent_ref[...] += ent_part

    @pl.when(i == nsteps - 1)
    def _fin():
        ent_ref[...] = -ent_ref[...] * (1.0 / N)


@jax.jit
def kernel(x, expert_bias, W):
    grid = (N // BN,)
    scores, tops, idx, counts, ent = pl.pallas_call(
        _router_body,
        grid=grid,
        in_specs=[
            pl.BlockSpec((BN, DIM), lambda i: (i, 0)),
            pl.BlockSpec((E, 1), lambda i: (0, 0)),
            pl.BlockSpec((E, DIM), lambda i: (0, 0)),
        ],
        out_specs=[
            pl.BlockSpec((BN, E), lambda i: (i, 0)),
            pl.BlockSpec((BN, K), lambda i: (i, 0)),
            pl.BlockSpec((BN, K), lambda i: (i, 0)),
            pl.BlockSpec((E, 1), lambda i: (0, 0)),
            pl.BlockSpec((1, 1), lambda i: (0, 0)),
        ],
        out_shape=[
            jax.ShapeDtypeStruct((N, E), jnp.float32),
            jax.ShapeDtypeStruct((N, K), jnp.float32),
            jax.ShapeDtypeStruct((N, K), jnp.int32),
            jax.ShapeDtypeStruct((E, 1), jnp.float32),
            jax.ShapeDtypeStruct((1, 1), jnp.float32),
        ],
    )(x, expert_bias.reshape(E, 1), W)
    return (tops, scores, idx, counts.reshape(E), ent.reshape(()))
